# baseline (speedup 1.0000x reference)
import dataclasses
import functools

import jax
import jax.numpy as jnp
from jax import lax
from jax.experimental import pallas as pl
from jax.experimental.pallas import tpu as pltpu
from jax.experimental.pallas import tpu_sc as plsc

N = 10000
D = 128
NE = 320000
NED = NE // 2
NRALL = 201
BATCH = 4096
NS = 16
NC = 2
B = 64
NB = 160
CH = 4
NCH = NB // CH
GB = 128
EPT = NB * B
E_PAD = NS * EPT
DB = 128
DNB = E_PAD // (NS * DB)
RPT = 640
N_P = NS * RPT
XS_ROWS = NC * N_P
NRALL_P = 208
BR = 2048
NSTEP = N_P // BR

_f32 = jnp.float32
_i32 = jnp.int32
_HP = jax.lax.Precision.HIGHEST
_mesh = plsc.VectorSubcoreMesh(core_axis_name="c", subcore_axis_name="s")
_sc_params = pltpu.CompilerParams()
if "needs_layout_passes" in pltpu.CompilerParams.__dataclass_fields__:
    _sc_params = dataclasses.replace(_sc_params, needs_layout_passes=False)


def _sds(shape, dtype=_f32):
    return jax.ShapeDtypeStruct(shape, dtype)


@functools.partial(
    pl.kernel,
    out_type=_sds((NC, N_P)),
    mesh=_mesh,
    scratch_types=[
        pltpu.VMEM((DB,), _i32),
        pltpu.VMEM((N_P,), _f32),
        pltpu.VMEM((NS, RPT), _f32),
        pltpu.VMEM_SHARED((NS, N_P), _f32),
    ],
    compiler_params=_sc_params,
)
def _deg_kernel(didx_hbm, deg_hbm, idx_v, hist, red, sbuf):
    cid = lax.axis_index("c")
    sid = lax.axis_index("s")
    ones16 = jnp.full((16,), 1.0, _f32)

    @pl.loop(0, N_P // 16)
    def _(i):
        hist[pl.ds(i * 16, 16)] = jnp.zeros((16,), _f32)

    @pl.loop(0, DNB)
    def _(b):
        base = sid * (DNB * DB) + b * DB
        pltpu.sync_copy(didx_hbm.at[cid, pl.ds(base, DB)], idx_v)
        for g in range(DB // 16):
            idx16 = idx_v[pl.ds(g * 16, 16)]
            plsc.addupdate_scatter(hist, [idx16], ones16)

    pltpu.sync_copy(hist, sbuf.at[sid])
    plsc.subcore_barrier()
    pltpu.sync_copy(sbuf.at[:, pl.ds(sid * RPT, RPT)], red)

    @pl.loop(0, RPT // 16)
    def _(i):
        sl = pl.ds(i * 16, 16)
        acc16 = red[0, sl]
        for t in range(1, NS):
            acc16 = acc16 + red[t, sl]
        hist[sl] = acc16

    pltpu.sync_copy(hist.at[pl.ds(0, RPT)],
                    deg_hbm.at[cid, pl.ds(sid * RPT, RPT)])


@functools.partial(
    pl.kernel,
    out_type=_sds((NC, N_P, D)),
    mesh=_mesh,
    scratch_types=[
        pltpu.VMEM((CH, 3, B), _i32),
        pltpu.VMEM((CH, 3, B), _i32),
        pltpu.VMEM((B, D), _f32),
        pltpu.VMEM((B, D), _f32),
        pltpu.VMEM((NRALL_P * D,), _f32),
        pltpu.VMEM((CH * B,), _i32),
        pltpu.VMEM((CH * B,), _i32),
        pltpu.VMEM_SHARED((N_P, D), _f32),
        pltpu.SemaphoreType.DMA,
        pltpu.SemaphoreType.DMA,
        pltpu.SemaphoreType.DMA,
        pltpu.SemaphoreType.DMA,
        pltpu.SemaphoreType.DMA,
        pltpu.SemaphoreType.DMA,
    ],
    compiler_params=_sc_params,
)
def _msg_kernel(xs_hbm, rel_hbm, idx_hbm, et_hbm, acc_hbm,
                idx_c0, idx_c1, xrows0, xrows1, rel_v, et_c0, et_c1, acc,
                semg0, semg1, sems0, sems1, semi0, semi1):
    cid = lax.axis_index("c")
    sid = lax.axis_index("s")
    xr = (xrows0, xrows1)
    ic = (idx_c0, idx_c1)
    ec = (et_c0, et_c1)
    semg = (semg0, semg1)
    sems = (sems0, sems1)
    semi = (semi0, semi1)

    pltpu.sync_copy(rel_hbm, rel_v)

    @pl.loop(0, B)
    def _(i):
        for c in range(D // 16):
            xrows0[i, pl.ds(c * 16, 16)] = jnp.zeros((16,), _f32)

    for k in range(RPT // B):
        pltpu.sync_copy(xrows0, acc.at[pl.ds(sid * RPT + k * B, B)])
    plsc.subcore_barrier()

    def issue_idx(c, h):
        pltpu.async_copy(idx_hbm.at[cid, sid, pl.ds(c * CH, CH)], ic[h],
                         semi[h])
        pltpu.async_copy(et_hbm.at[cid, sid, c], ec[h], semi[h])

    def wait_idx(h):
        pltpu.make_async_copy(idx_hbm.at[cid, sid, pl.ds(0, CH)], ic[h],
                              semi[h]).wait()
        pltpu.make_async_copy(et_hbm.at[cid, sid, 0], ec[h], semi[h]).wait()

    def start_gather(idx_c, k, p):
        pltpu.async_copy(xs_hbm.at[idx_c.at[k, 0]], xr[p], semg[p])

    def wait_gather(p):
        pltpu.make_async_copy(xs_hbm.at[pl.ds(0, B)], xr[p], semg[p]).wait()

    def wait_scatter(p):
        pltpu.make_async_copy(xr[p], acc.at[pl.ds(0, B)], sems[p]).wait()

    issue_idx(0, 0)

    @pl.loop(0, NCH, step=2)
    def _(ch):
        for half in (0, 1):
            cc = ch + half
            wait_idx(half)
            nxt = cc + 1

            @pl.when(nxt < NCH)
            def _():
                issue_idx(nxt, (half + 1) % 2)

            idx_c = ic[half]
            et_c = ec[half]
            start_gather(idx_c, 0, 0)
            for k in range(CH):
                p = k % 2
                q = (k + 1) % 2
                if k + 1 < CH:
                    if k >= 1:
                        wait_scatter(q)
                    start_gather(idx_c, k + 1, q)
                wait_gather(p)
                for g in range(0, B, 16):
                    et16 = et_c[pl.ds(k * B + g, 16)]

                    @plsc.parallel_loop(0, 16, unroll=8)
                    def _(j):
                        e = g + j
                        et_b = lax.gather(
                            et16, jnp.full((16, 1), j, _i32),
                            lax.GatherDimensionNumbers(
                                offset_dims=(), collapsed_slice_dims=(0,),
                                start_index_map=(0,)),
                            (1,),
                            mode=lax.GatherScatterMode.PROMISE_IN_BOUNDS)
                        base_a = et_b * D
                        for c in range(D // 16):
                            addr = base_a + (c * 16 + lax.iota(_i32, 16))
                            rrow = plsc.load_gather(rel_v, [addr])
                            sl = pl.ds(c * 16, 16)
                            xr[p][e, sl] = xr[p][e, sl] * rrow

                pltpu.async_copy(xr[p], acc.at[idx_c.at[k, 2]], sems[p],
                                 add=True)
            wait_scatter(0)
            wait_scatter(1)

    plsc.subcore_barrier()
    for k in range(RPT // B):
        r0 = sid * RPT + k * B
        pltpu.sync_copy(acc.at[pl.ds(r0, B)], acc_hbm.at[cid, pl.ds(r0, B)])


@functools.partial(
    pl.kernel,
    out_type=[_sds((BATCH, D)), _sds((BATCH, D)), _sds((BATCH, D))],
    mesh=_mesh,
    scratch_types=[
        pltpu.VMEM((GB,), _i32),
        pltpu.VMEM((GB, D), _f32),
        pltpu.SemaphoreType.DMA,
    ],
)
def _gather_kernel(x_hbm, r_hbm, sub_hbm, rel_hbm, obj_hbm,
                   sub_o, rel_o, obj_o, idx_v, rows_v, sem):
    cid = lax.axis_index("c")
    sid = lax.axis_index("s")
    base = (sid * NC + cid) * GB
    for tab, ih, oh in ((x_hbm, sub_hbm, sub_o),
                        (r_hbm, rel_hbm, rel_o),
                        (x_hbm, obj_hbm, obj_o)):
        pltpu.sync_copy(ih.at[pl.ds(base, GB)], idx_v)
        pltpu.async_copy(tab.at[idx_v], rows_v, sem).wait()
        pltpu.sync_copy(rows_v, oh.at[pl.ds(base, GB)])


def _deg_inv(deg):
    return jnp.where(deg > 0.0, 1.0 / jnp.sqrt(jnp.maximum(deg, 1e-12)), 0.0)


def _dot(a, b):
    return jnp.dot(a, b, preferred_element_type=_f32, precision=_HP)


def _prep_body(x_ref, deg_ref, relwt_ref, initrel_ref, looprel_ref,
               xs_ref, relall_ref):
    i = pl.program_id(0)
    di = _deg_inv(deg_ref[0, :])
    do = _deg_inv(deg_ref[1, :])
    x = x_ref[...]
    xs_ref[0, :, :] = x * di[:, None]
    xs_ref[1, :, :] = x * do[:, None]

    @pl.when(i == 0)
    def _():
        rel = _dot(relwt_ref[...], initrel_ref[...])
        relall_ref[...] = jnp.concatenate([rel, looprel_ref[...]], axis=0)


_prep = pl.pallas_call(
    _prep_body,
    grid=(NSTEP,),
    in_specs=[
        pl.BlockSpec((BR, D), lambda i: (i, 0)),
        pl.BlockSpec((2, BR), lambda i: (0, i)),
        pl.BlockSpec((NRALL - 1, 5), lambda i: (0, 0)),
        pl.BlockSpec((5, D), lambda i: (0, 0)),
        pl.BlockSpec((1, D), lambda i: (0, 0)),
    ],
    out_specs=[
        pl.BlockSpec((2, BR, D), lambda i: (0, i, 0)),
        pl.BlockSpec((NRALL, D), lambda i: (0, 0)),
    ],
    out_shape=[_sds((NC, N_P, D)), _sds((NRALL, D))],
)


def _conv_body(final, acc_ref, x_ref, deg_ref, w_in_ref, w_out_ref,
               w_loop_ref, relall_ref, w_rel_ref, gamma_ref, beta_ref,
               looprel_next_ref, *rest):
    if final:
        x_out_ref, r_out_ref, stats = rest
    else:
        x_out_ref, xs_ref, relall2_ref, stats = rest
    p = pl.program_id(0)
    i = pl.program_id(1)
    di = _deg_inv(deg_ref[0, :])
    do = _deg_inv(deg_ref[1, :])
    x = x_ref[...]
    loop_row = relall_ref[NRALL - 1:NRALL, :]
    out = (_dot(acc_ref[0, :, :] * di[:, None], w_in_ref[...])
           + _dot(acc_ref[1, :, :] * do[:, None], w_out_ref[...])
           + _dot(x * loop_row, w_loop_ref[...])) / 3.0

    @pl.when(p == 0)
    def _():
        @pl.when(i == 0)
        def _():
            stats[...] = jnp.zeros((8, D), _f32)

        stats[0, :] += jnp.sum(out, axis=0)
        stats[1, :] += jnp.sum(out * out, axis=0)

    @pl.when(p == 1)
    def _():
        mean = stats[0, :] / float(N)
        var = stats[1, :] / float(N) - mean * mean
        xn = (out - mean[None, :]) / jnp.sqrt(var[None, :] + 1e-5)
        xnew = jnp.tanh(xn * gamma_ref[...][None, :] + beta_ref[...][None, :])
        rows = i * BR + lax.broadcasted_iota(_i32, (BR, 1), 0)
        xnew = jnp.where(rows < N, xnew, 0.0)
        x_out_ref[...] = xnew
        if not final:
            xs_ref[0, :, :] = xnew * di[:, None]
            xs_ref[1, :, :] = xnew * do[:, None]

        @pl.when(i == 0)
        def _():
            rel_next = _dot(relall_ref[...], w_rel_ref[...])
            if final:
                r_out_ref[...] = rel_next[:NRALL - 1, :]
            else:
                relall2_ref[...] = jnp.concatenate(
                    [rel_next[:NRALL - 1, :], looprel_next_ref[...]], axis=0)


def _make_conv(final):
    in_specs = [
        pl.BlockSpec((2, BR, D), lambda p, i: (0, i, 0)),
        pl.BlockSpec((BR, D), lambda p, i: (i, 0)),
        pl.BlockSpec((2, BR), lambda p, i: (0, i)),
        pl.BlockSpec((D, D), lambda p, i: (0, 0)),
        pl.BlockSpec((D, D), lambda p, i: (0, 0)),
        pl.BlockSpec((D, D), lambda p, i: (0, 0)),
        pl.BlockSpec((NRALL, D), lambda p, i: (0, 0)),
        pl.BlockSpec((D, D), lambda p, i: (0, 0)),
        pl.BlockSpec((D,), lambda p, i: (0,)),
        pl.BlockSpec((D,), lambda p, i: (0,)),
        pl.BlockSpec((1, D), lambda p, i: (0, 0)),
    ]
    if final:
        out_specs = [
            pl.BlockSpec((BR, D), lambda p, i: (i, 0)),
            pl.BlockSpec((NRALL - 1, D), lambda p, i: (0, 0)),
        ]
        out_shape = [_sds((N_P, D)), _sds((NRALL - 1, D))]
    else:
        out_specs = [
            pl.BlockSpec((BR, D), lambda p, i: (i, 0)),
            pl.BlockSpec((2, BR, D), lambda p, i: (0, i, 0)),
            pl.BlockSpec((NRALL, D), lambda p, i: (0, 0)),
        ]
        out_shape = [_sds((N_P, D)), _sds((NC, N_P, D)), _sds((NRALL, D))]
    return pl.pallas_call(
        functools.partial(_conv_body, final),
        grid=(2, NSTEP),
        in_specs=in_specs,
        out_specs=out_specs,
        out_shape=out_shape,
        scratch_shapes=[pltpu.VMEM((8, D), _f32)],
    )


_conv_mid = _make_conv(False)
_conv_fin = _make_conv(True)


def kernel(sub, rel, obj, edge_index, edge_type, init_embed, init_rel,
           rel_wt1, w_in1, w_out1, w_loop1, w_rel1, loop_rel1, gamma1, beta1,
           w_in2, w_out2, w_loop2, w_rel2, loop_rel2, gamma2, beta2):
    ei = edge_index.astype(_i32)
    et = edge_type.astype(_i32)
    npad = E_PAD - NED

    def _pad(a, v):
        return jnp.concatenate([a, jnp.full((npad,), v, _i32)])

    in_src = _pad(ei[0, :NED], N)
    in_dst = _pad(ei[1, :NED], N)
    in_t = _pad(et[:NED], 0)
    out_src = _pad(ei[0, NED:], N)
    out_dst = _pad(ei[1, NED:], N)
    out_t = _pad(et[NED:], 0)

    didx2 = jnp.stack([in_src, out_src])

    def _blockify(src, t, dst):
        a = jnp.stack([src, t, dst], axis=0)
        return a.reshape(3, NS, NB, B).transpose(1, 2, 0, 3)

    idx_all = jnp.stack([_blockify(in_src, in_t, in_dst),
                         _blockify(out_src + N_P, out_t, out_dst)])
    et_all = jnp.stack([in_t.reshape(NS, NCH, CH * B),
                        out_t.reshape(NS, NCH, CH * B)])

    x0 = jnp.concatenate([init_embed, jnp.zeros((N_P - N, D), _f32)], axis=0)

    deg2 = _deg_kernel(didx2)
    xs1, relall1 = _prep(x0, deg2, rel_wt1, init_rel, loop_rel1)
    relpad = jnp.zeros((NRALL_P - NRALL, D), _f32)
    acc1 = _msg_kernel(xs1.reshape(XS_ROWS, D),
                       jnp.concatenate([relall1, relpad]).reshape(NRALL_P * D),
                       idx_all, et_all)
    x1, xs2, relall2 = _conv_mid(acc1, x0, deg2, w_in1, w_out1, w_loop1,
                                 relall1, w_rel1, gamma1, beta1, loop_rel2)
    acc2 = _msg_kernel(xs2.reshape(XS_ROWS, D),
                       jnp.concatenate([relall2, relpad]).reshape(NRALL_P * D),
                       idx_all, et_all)
    x2, r_out = _conv_fin(acc2, x1, deg2, w_in2, w_out2, w_loop2, relall2,
                          w_rel2, gamma2, beta2, loop_rel2)
    return tuple(_gather_kernel(x2, r_out, sub.astype(_i32), rel.astype(_i32),
                                obj.astype(_i32)))


# speedup vs baseline: 1.0330x; 1.0330x over previous
import dataclasses
import functools

import jax
import jax.numpy as jnp
from jax import lax
from jax.experimental import pallas as pl
from jax.experimental.pallas import tpu as pltpu
from jax.experimental.pallas import tpu_sc as plsc

N = 10000
D = 128
NE = 320000
NED = NE // 2
NRALL = 201
BATCH = 4096
NS = 16
NC = 2
B = 64
NB = 160
CH = 8
NCH = NB // CH
GB = 128
EPT = NB * B
E_PAD = NS * EPT
DB = 128
DNB = E_PAD // (NS * DB)
RPT = 640
N_P = NS * RPT
XS_ROWS = NC * N_P
NRALL_P = 208
BR = 2048
NSTEP = N_P // BR

_f32 = jnp.float32
_i32 = jnp.int32
_HP = jax.lax.Precision.HIGHEST
_mesh = plsc.VectorSubcoreMesh(core_axis_name="c", subcore_axis_name="s")
_sc_params = pltpu.CompilerParams()
if "needs_layout_passes" in pltpu.CompilerParams.__dataclass_fields__:
    _sc_params = dataclasses.replace(_sc_params, needs_layout_passes=False)


def _sds(shape, dtype=_f32):
    return jax.ShapeDtypeStruct(shape, dtype)


@functools.partial(
    pl.kernel,
    out_type=_sds((NC, N_P)),
    mesh=_mesh,
    scratch_types=[
        pltpu.VMEM((DB,), _i32),
        pltpu.VMEM((N_P,), _f32),
        pltpu.VMEM((NS, RPT), _f32),
        pltpu.VMEM_SHARED((NS, N_P), _f32),
    ],
    compiler_params=_sc_params,
)
def _deg_kernel(didx_hbm, deg_hbm, idx_v, hist, red, sbuf):
    cid = lax.axis_index("c")
    sid = lax.axis_index("s")
    ones16 = jnp.full((16,), 1.0, _f32)

    @pl.loop(0, N_P // 16)
    def _(i):
        hist[pl.ds(i * 16, 16)] = jnp.zeros((16,), _f32)

    @pl.loop(0, DNB)
    def _(b):
        base = sid * (DNB * DB) + b * DB
        pltpu.sync_copy(didx_hbm.at[cid, pl.ds(base, DB)], idx_v)
        for g in range(DB // 16):
            idx16 = idx_v[pl.ds(g * 16, 16)]
            plsc.addupdate_scatter(hist, [idx16], ones16)

    pltpu.sync_copy(hist, sbuf.at[sid])
    plsc.subcore_barrier()
    pltpu.sync_copy(sbuf.at[:, pl.ds(sid * RPT, RPT)], red)

    @pl.loop(0, RPT // 16)
    def _(i):
        sl = pl.ds(i * 16, 16)
        acc16 = red[0, sl]
        for t in range(1, NS):
            acc16 = acc16 + red[t, sl]
        hist[sl] = acc16

    pltpu.sync_copy(hist.at[pl.ds(0, RPT)],
                    deg_hbm.at[cid, pl.ds(sid * RPT, RPT)])


@functools.partial(
    pl.kernel,
    out_type=_sds((NC, N_P, D)),
    mesh=_mesh,
    scratch_types=[
        pltpu.VMEM((CH, 3, B), _i32),
        pltpu.VMEM((B, D), _f32),
        pltpu.VMEM((B, D), _f32),
        pltpu.VMEM((NRALL_P * D,), _f32),
        pltpu.VMEM((CH * B,), _i32),
        pltpu.VMEM_SHARED((N_P, D), _f32),
        pltpu.SemaphoreType.DMA,
        pltpu.SemaphoreType.DMA,
        pltpu.SemaphoreType.DMA,
        pltpu.SemaphoreType.DMA,
    ],
    compiler_params=_sc_params,
)
def _msg_kernel(xs_hbm, rel_hbm, idx_hbm, et_hbm, acc_hbm,
                idx_c, xrows0, xrows1, rel_v, et_c, acc,
                semg0, semg1, sems0, sems1):
    cid = lax.axis_index("c")
    sid = lax.axis_index("s")
    xr = (xrows0, xrows1)
    semg = (semg0, semg1)
    sems = (sems0, sems1)

    pltpu.sync_copy(rel_hbm, rel_v)

    @pl.loop(0, B)
    def _(i):
        for c in range(D // 16):
            xrows0[i, pl.ds(c * 16, 16)] = jnp.zeros((16,), _f32)

    for k in range(RPT // B):
        pltpu.sync_copy(xrows0, acc.at[pl.ds(sid * RPT + k * B, B)])
    plsc.subcore_barrier()

    def start_gather(k, p):
        pltpu.async_copy(xs_hbm.at[idx_c.at[k, 0]], xr[p], semg[p])

    def wait_gather(p):
        pltpu.make_async_copy(xs_hbm.at[pl.ds(0, B)], xr[p], semg[p]).wait()

    def wait_scatter(p):
        pltpu.make_async_copy(xr[p], acc.at[pl.ds(0, B)], sems[p]).wait()

    @pl.loop(0, NCH)
    def _(ch):
        pltpu.sync_copy(idx_hbm.at[cid, sid, pl.ds(ch * CH, CH)], idx_c)
        pltpu.sync_copy(et_hbm.at[cid, sid, ch], et_c)
        start_gather(0, 0)
        for k in range(CH):
            p = k % 2
            q = (k + 1) % 2
            if k + 1 < CH:
                if k >= 1:
                    wait_scatter(q)
                start_gather(k + 1, q)
            wait_gather(p)
            for g in range(0, B, 16):
                et16 = et_c[pl.ds(k * B + g, 16)]

                @plsc.parallel_loop(0, 16, unroll=8)
                def _(j):
                    e = g + j
                    et_b = lax.gather(
                        et16, jnp.full((16, 1), j, _i32),
                        lax.GatherDimensionNumbers(
                            offset_dims=(), collapsed_slice_dims=(0,),
                            start_index_map=(0,)),
                        (1,), mode=lax.GatherScatterMode.PROMISE_IN_BOUNDS)
                    base_a = et_b * D
                    for c in range(D // 16):
                        addr = base_a + (c * 16 + lax.iota(_i32, 16))
                        rrow = plsc.load_gather(rel_v, [addr])
                        sl = pl.ds(c * 16, 16)
                        xr[p][e, sl] = xr[p][e, sl] * rrow

            pltpu.async_copy(xr[p], acc.at[idx_c.at[k, 2]], sems[p],
                             add=True)
        wait_scatter(0)
        wait_scatter(1)

    plsc.subcore_barrier()
    for k in range(RPT // B):
        r0 = sid * RPT + k * B
        pltpu.sync_copy(acc.at[pl.ds(r0, B)], acc_hbm.at[cid, pl.ds(r0, B)])


@functools.partial(
    pl.kernel,
    out_type=[_sds((BATCH, D)), _sds((BATCH, D)), _sds((BATCH, D))],
    mesh=_mesh,
    scratch_types=[
        pltpu.VMEM((GB,), _i32),
        pltpu.VMEM((GB, D), _f32),
        pltpu.SemaphoreType.DMA,
    ],
)
def _gather_kernel(x_hbm, r_hbm, sub_hbm, rel_hbm, obj_hbm,
                   sub_o, rel_o, obj_o, idx_v, rows_v, sem):
    cid = lax.axis_index("c")
    sid = lax.axis_index("s")
    base = (sid * NC + cid) * GB
    for tab, ih, oh in ((x_hbm, sub_hbm, sub_o),
                        (r_hbm, rel_hbm, rel_o),
                        (x_hbm, obj_hbm, obj_o)):
        pltpu.sync_copy(ih.at[pl.ds(base, GB)], idx_v)
        pltpu.async_copy(tab.at[idx_v], rows_v, sem).wait()
        pltpu.sync_copy(rows_v, oh.at[pl.ds(base, GB)])


def _deg_inv(deg):
    return jnp.where(deg > 0.0, 1.0 / jnp.sqrt(jnp.maximum(deg, 1e-12)), 0.0)


def _dot(a, b):
    return jnp.dot(a, b, preferred_element_type=_f32, precision=_HP)


def _prep_body(x_ref, deg_ref, relwt_ref, initrel_ref, looprel_ref,
               xs_ref, relall_ref):
    i = pl.program_id(0)
    di = _deg_inv(deg_ref[0, :])
    do = _deg_inv(deg_ref[1, :])
    x = x_ref[...]
    xs_ref[0, :, :] = x * di[:, None]
    xs_ref[1, :, :] = x * do[:, None]

    @pl.when(i == 0)
    def _():
        rel = _dot(relwt_ref[...], initrel_ref[...])
        relall_ref[...] = jnp.concatenate([rel, looprel_ref[...]], axis=0)


_prep = pl.pallas_call(
    _prep_body,
    grid=(NSTEP,),
    in_specs=[
        pl.BlockSpec((BR, D), lambda i: (i, 0)),
        pl.BlockSpec((2, BR), lambda i: (0, i)),
        pl.BlockSpec((NRALL - 1, 5), lambda i: (0, 0)),
        pl.BlockSpec((5, D), lambda i: (0, 0)),
        pl.BlockSpec((1, D), lambda i: (0, 0)),
    ],
    out_specs=[
        pl.BlockSpec((2, BR, D), lambda i: (0, i, 0)),
        pl.BlockSpec((NRALL, D), lambda i: (0, 0)),
    ],
    out_shape=[_sds((NC, N_P, D)), _sds((NRALL, D))],
)


def _conv_body(final, acc_ref, x_ref, deg_ref, w_in_ref, w_out_ref,
               w_loop_ref, relall_ref, w_rel_ref, gamma_ref, beta_ref,
               looprel_next_ref, *rest):
    if final:
        x_out_ref, r_out_ref, stats = rest
    else:
        x_out_ref, xs_ref, relall2_ref, stats = rest
    p = pl.program_id(0)
    i = pl.program_id(1)
    di = _deg_inv(deg_ref[0, :])
    do = _deg_inv(deg_ref[1, :])
    x = x_ref[...]
    loop_row = relall_ref[NRALL - 1:NRALL, :]
    out = (_dot(acc_ref[0, :, :] * di[:, None], w_in_ref[...])
           + _dot(acc_ref[1, :, :] * do[:, None], w_out_ref[...])
           + _dot(x * loop_row, w_loop_ref[...])) / 3.0

    @pl.when(p == 0)
    def _():
        @pl.when(i == 0)
        def _():
            stats[...] = jnp.zeros((8, D), _f32)

        stats[0, :] += jnp.sum(out, axis=0)
        stats[1, :] += jnp.sum(out * out, axis=0)

    @pl.when(p == 1)
    def _():
        mean = stats[0, :] / float(N)
        var = stats[1, :] / float(N) - mean * mean
        xn = (out - mean[None, :]) / jnp.sqrt(var[None, :] + 1e-5)
        xnew = jnp.tanh(xn * gamma_ref[...][None, :] + beta_ref[...][None, :])
        rows = i * BR + lax.broadcasted_iota(_i32, (BR, 1), 0)
        xnew = jnp.where(rows < N, xnew, 0.0)
        x_out_ref[...] = xnew
        if not final:
            xs_ref[0, :, :] = xnew * di[:, None]
            xs_ref[1, :, :] = xnew * do[:, None]

        @pl.when(i == 0)
        def _():
            rel_next = _dot(relall_ref[...], w_rel_ref[...])
            if final:
                r_out_ref[...] = rel_next[:NRALL - 1, :]
            else:
                relall2_ref[...] = jnp.concatenate(
                    [rel_next[:NRALL - 1, :], looprel_next_ref[...]], axis=0)


def _make_conv(final):
    in_specs = [
        pl.BlockSpec((2, BR, D), lambda p, i: (0, i, 0)),
        pl.BlockSpec((BR, D), lambda p, i: (i, 0)),
        pl.BlockSpec((2, BR), lambda p, i: (0, i)),
        pl.BlockSpec((D, D), lambda p, i: (0, 0)),
        pl.BlockSpec((D, D), lambda p, i: (0, 0)),
        pl.BlockSpec((D, D), lambda p, i: (0, 0)),
        pl.BlockSpec((NRALL, D), lambda p, i: (0, 0)),
        pl.BlockSpec((D, D), lambda p, i: (0, 0)),
        pl.BlockSpec((D,), lambda p, i: (0,)),
        pl.BlockSpec((D,), lambda p, i: (0,)),
        pl.BlockSpec((1, D), lambda p, i: (0, 0)),
    ]
    if final:
        out_specs = [
            pl.BlockSpec((BR, D), lambda p, i: (i, 0)),
            pl.BlockSpec((NRALL - 1, D), lambda p, i: (0, 0)),
        ]
        out_shape = [_sds((N_P, D)), _sds((NRALL - 1, D))]
    else:
        out_specs = [
            pl.BlockSpec((BR, D), lambda p, i: (i, 0)),
            pl.BlockSpec((2, BR, D), lambda p, i: (0, i, 0)),
            pl.BlockSpec((NRALL, D), lambda p, i: (0, 0)),
        ]
        out_shape = [_sds((N_P, D)), _sds((NC, N_P, D)), _sds((NRALL, D))]
    return pl.pallas_call(
        functools.partial(_conv_body, final),
        grid=(2, NSTEP),
        in_specs=in_specs,
        out_specs=out_specs,
        out_shape=out_shape,
        scratch_shapes=[pltpu.VMEM((8, D), _f32)],
    )


_conv_mid = _make_conv(False)
_conv_fin = _make_conv(True)


def kernel(sub, rel, obj, edge_index, edge_type, init_embed, init_rel,
           rel_wt1, w_in1, w_out1, w_loop1, w_rel1, loop_rel1, gamma1, beta1,
           w_in2, w_out2, w_loop2, w_rel2, loop_rel2, gamma2, beta2):
    ei = edge_index.astype(_i32)
    et = edge_type.astype(_i32)
    npad = E_PAD - NED

    def _pad(a, v):
        return jnp.concatenate([a, jnp.full((npad,), v, _i32)])

    in_src = _pad(ei[0, :NED], N)
    in_dst = _pad(ei[1, :NED], N)
    in_t = _pad(et[:NED], 0)
    out_src = _pad(ei[0, NED:], N)
    out_dst = _pad(ei[1, NED:], N)
    out_t = _pad(et[NED:], 0)

    didx2 = jnp.stack([in_src, out_src])

    def _blockify(src, t, dst):
        a = jnp.stack([src, t, dst], axis=0)
        return a.reshape(3, NS, NB, B).transpose(1, 2, 0, 3)

    idx_all = jnp.stack([_blockify(in_src, in_t, in_dst),
                         _blockify(out_src + N_P, out_t, out_dst)])
    et_all = jnp.stack([in_t.reshape(NS, NCH, CH * B),
                        out_t.reshape(NS, NCH, CH * B)])

    x0 = jnp.concatenate([init_embed, jnp.zeros((N_P - N, D), _f32)], axis=0)

    deg2 = _deg_kernel(didx2)
    xs1, relall1 = _prep(x0, deg2, rel_wt1, init_rel, loop_rel1)
    relpad = jnp.zeros((NRALL_P - NRALL, D), _f32)
    acc1 = _msg_kernel(xs1.reshape(XS_ROWS, D),
                       jnp.concatenate([relall1, relpad]).reshape(NRALL_P * D),
                       idx_all, et_all)
    x1, xs2, relall2 = _conv_mid(acc1, x0, deg2, w_in1, w_out1, w_loop1,
                                 relall1, w_rel1, gamma1, beta1, loop_rel2)
    acc2 = _msg_kernel(xs2.reshape(XS_ROWS, D),
                       jnp.concatenate([relall2, relpad]).reshape(NRALL_P * D),
                       idx_all, et_all)
    x2, r_out = _conv_fin(acc2, x1, deg2, w_in2, w_out2, w_loop2, relall2,
                          w_rel2, gamma2, beta2, loop_rel2)
    return tuple(_gather_kernel(x2, r_out, sub.astype(_i32), rel.astype(_i32),
                                obj.astype(_i32)))
